# baseline (speedup 1.0000x reference)
import functools

import jax
import jax.numpy as jnp
from jax import lax
from jax.experimental import pallas as pl
from jax.experimental.pallas import tpu as pltpu
from jax.experimental.pallas import tpu_sc as plsc

_NBINS = 29
_NC, _NS, _L = 2, 16, 16
_NW = _NC * _NS
_LSTR = 33
_RSTR = 545
_QUAD = 8


@functools.partial(jax.jit, static_argnums=(3, 4))
def _hist_call(sim2, dtok, qtok, c_per_b, d):
    rows = sim2.shape[0]
    rpw = rows // _NW
    chunks = d // _L
    hsize = rpw * _RSTR

    mesh = plsc.VectorSubcoreMesh(core_axis_name="c", subcore_axis_name="s")

    @functools.partial(
        pl.kernel,
        mesh=mesh,
        compiler_params=pltpu.CompilerParams(needs_layout_passes=False),
        out_type=jax.ShapeDtypeStruct((rows, 32), jnp.float32),
        scratch_types=[
            pltpu.VMEM((rpw, d), jnp.float32),
            pltpu.VMEM((d,), jnp.int32),
            pltpu.VMEM((_L,), jnp.int32),
            pltpu.VMEM((d,), jnp.float32),
            pltpu.VMEM((_L,), jnp.float32),
            pltpu.VMEM((hsize,), jnp.float32),
            pltpu.VMEM((rpw, 32), jnp.float32),
            pltpu.SemaphoreType.DMA,
        ],
    )
    def body(sim_hbm, dtok_hbm, qtok_hbm, out_hbm, sim_v, dtok_v, qtok_v,
             wd_v, mq_v, hist_v, out_v, sem0):
        wid = lax.axis_index("s") * _NC + lax.axis_index("c")
        base = wid * rpw
        b = base // c_per_b

        cp0 = pltpu.async_copy(sim_hbm.at[pl.ds(base, rpw)], sim_v, sem0)
        pltpu.sync_copy(dtok_hbm.at[b], dtok_v)
        pltpu.sync_copy(qtok_hbm.at[b], qtok_v)

        zero = jnp.zeros((_L,), jnp.float32)
        one = jnp.ones((_L,), jnp.float32)

        @plsc.parallel_loop(0, hsize // _L, unroll=4)
        def _zbody(i):
            hist_v[pl.ds(i * _L, _L)] = zero

        @plsc.parallel_loop(0, chunks, unroll=4)
        def _wbody(i):
            t = dtok_v[pl.ds(i * _L, _L)]
            wd_v[pl.ds(i * _L, _L)] = jnp.where(t != jnp.int32(-1), one, zero)

        qt = qtok_v[pl.ds(0, _L)]
        mq_v[pl.ds(0, _L)] = jnp.where(qt != jnp.int32(-1), one, zero)

        lane = lax.iota(jnp.int32, _L) * _LSTR

        def quad_body(cq, carry):
            cb = cq * (_QUAD * _L)
            ws = [wd_v[pl.ds(cb + j * _L, _L)] for j in range(_QUAD)]

            @plsc.parallel_loop(0, rpw, unroll=2)
            def _rbody(r):
                lane_r = lane + r * _RSTR
                for j in range(_QUAD):
                    s = sim_v[r, pl.ds(cb + j * _L, _L)]
                    bins = ((s + 1.00001) * 14.0).astype(jnp.int32)
                    plsc.addupdate_scatter(hist_v, [bins + lane_r], ws[j])

            return carry

        cp0.wait()
        lax.fori_loop(0, chunks // _QUAD, quad_body, 0)

        def red_body(r, carry):
            @plsc.parallel_loop(
                0, _L, unroll=2,
                carry=(jnp.zeros((_L,), jnp.float32),
                       jnp.zeros((_L,), jnp.float32)))
            def _accs(ln, accs):
                a0, a1 = accs
                o = r * _RSTR + ln * _LSTR
                t0 = hist_v[pl.ds(o, _L)]
                t1 = hist_v[pl.ds(o + _L, _L)]
                hist_v[pl.ds(o, _L)] = zero
                hist_v[pl.ds(o + _L, _L)] = zero
                return (a0 + t0, a1 + t1)

            acc0, acc1 = _accs
            mq = plsc.load_gather(mq_v, [jnp.full((_L,), r % 16, jnp.int32)])
            out_v[r, pl.ds(0, _L)] = acc0 * mq
            out_v[r, pl.ds(_L, _L)] = acc1 * mq
            return carry

        lax.fori_loop(0, rpw, red_body, 0)

        pltpu.sync_copy(out_v, out_hbm.at[pl.ds(base, rpw)])

    return body(sim2, dtok, qtok)


def kernel(simmat, dlens, dtoks, qtoks):
    del dlens
    B, C, Q, D = simmat.shape
    sim2 = simmat.reshape(B * C * Q, D)
    dtok = dtoks.astype(jnp.int32)
    qtok = qtoks.astype(jnp.int32)
    out = _hist_call(sim2, dtok, qtok, C * Q, D)
    return out[:, :_NBINS].reshape(B, C, Q, _NBINS)


# speedup vs baseline: 1.0066x; 1.0066x over previous
import functools

import jax
import jax.numpy as jnp
from jax import lax
from jax.experimental import pallas as pl
from jax.experimental.pallas import tpu as pltpu
from jax.experimental.pallas import tpu_sc as plsc

_NBINS = 29
_NC, _NS, _L = 2, 16, 16
_NW = _NC * _NS
_LSTR = 33
_RSTR = 545
_QUAD = 4


@functools.partial(jax.jit, static_argnums=(3, 4))
def _hist_call(sim2, dtok, qtok, c_per_b, d):
    rows = sim2.shape[0]
    rpw = rows // _NW
    chunks = d // _L
    hsize = rpw * _RSTR

    mesh = plsc.VectorSubcoreMesh(core_axis_name="c", subcore_axis_name="s")

    @functools.partial(
        pl.kernel,
        mesh=mesh,
        compiler_params=pltpu.CompilerParams(needs_layout_passes=False),
        out_type=jax.ShapeDtypeStruct((rows, 32), jnp.float32),
        scratch_types=[
            pltpu.VMEM((rpw, d), jnp.float32),
            pltpu.VMEM((d,), jnp.int32),
            pltpu.VMEM((_L,), jnp.int32),
            pltpu.VMEM((d,), jnp.float32),
            pltpu.VMEM((_L,), jnp.float32),
            pltpu.VMEM((hsize,), jnp.float32),
            pltpu.VMEM((rpw, 32), jnp.float32),
            pltpu.SemaphoreType.DMA,
        ],
    )
    def body(sim_hbm, dtok_hbm, qtok_hbm, out_hbm, sim_v, dtok_v, qtok_v,
             wd_v, mq_v, hist_v, out_v, sem0):
        wid = lax.axis_index("s") * _NC + lax.axis_index("c")
        base = wid * rpw
        b = base // c_per_b

        cp0 = pltpu.async_copy(sim_hbm.at[pl.ds(base, rpw)], sim_v, sem0)
        pltpu.sync_copy(dtok_hbm.at[b], dtok_v)
        pltpu.sync_copy(qtok_hbm.at[b], qtok_v)

        zero = jnp.zeros((_L,), jnp.float32)
        one = jnp.ones((_L,), jnp.float32)

        @plsc.parallel_loop(0, hsize // _L, unroll=4)
        def _zbody(i):
            hist_v[pl.ds(i * _L, _L)] = zero

        @plsc.parallel_loop(0, chunks, unroll=4)
        def _wbody(i):
            t = dtok_v[pl.ds(i * _L, _L)]
            wd_v[pl.ds(i * _L, _L)] = jnp.where(t != jnp.int32(-1), one, zero)

        qt = qtok_v[pl.ds(0, _L)]
        mq_v[pl.ds(0, _L)] = jnp.where(qt != jnp.int32(-1), one, zero)

        lane = lax.iota(jnp.int32, _L) * _LSTR

        def quad_body(cq, carry):
            cb = cq * (_QUAD * _L)
            ws = [wd_v[pl.ds(cb + j * _L, _L)] for j in range(_QUAD)]

            @plsc.parallel_loop(0, rpw, unroll=4)
            def _rbody(r):
                lane_r = lane + r * _RSTR
                for j in range(_QUAD):
                    s = sim_v[r, pl.ds(cb + j * _L, _L)]
                    bins = ((s + 1.00001) * 14.0).astype(jnp.int32)
                    plsc.addupdate_scatter(hist_v, [bins + lane_r], ws[j])

            return carry

        cp0.wait()
        lax.fori_loop(0, chunks // _QUAD, quad_body, 0)

        def red_body(r, carry):
            @plsc.parallel_loop(
                0, _L, unroll=2,
                carry=(jnp.zeros((_L,), jnp.float32),
                       jnp.zeros((_L,), jnp.float32)))
            def _accs(ln, accs):
                a0, a1 = accs
                o = r * _RSTR + ln * _LSTR
                t0 = hist_v[pl.ds(o, _L)]
                t1 = hist_v[pl.ds(o + _L, _L)]
                hist_v[pl.ds(o, _L)] = zero
                hist_v[pl.ds(o + _L, _L)] = zero
                return (a0 + t0, a1 + t1)

            acc0, acc1 = _accs
            mq = plsc.load_gather(mq_v, [jnp.full((_L,), r % 16, jnp.int32)])
            out_v[r, pl.ds(0, _L)] = acc0 * mq
            out_v[r, pl.ds(_L, _L)] = acc1 * mq
            return carry

        lax.fori_loop(0, rpw, red_body, 0)

        pltpu.sync_copy(out_v, out_hbm.at[pl.ds(base, rpw)])

    return body(sim2, dtok, qtok)


def kernel(simmat, dlens, dtoks, qtoks):
    del dlens
    B, C, Q, D = simmat.shape
    sim2 = simmat.reshape(B * C * Q, D)
    dtok = dtoks.astype(jnp.int32)
    qtok = qtoks.astype(jnp.int32)
    out = _hist_call(sim2, dtok, qtok, C * Q, D)
    return out[:, :_NBINS].reshape(B, C, Q, _NBINS)


# speedup vs baseline: 1.0143x; 1.0077x over previous
import functools

import jax
import jax.numpy as jnp
from jax import lax
from jax.experimental import pallas as pl
from jax.experimental.pallas import tpu as pltpu
from jax.experimental.pallas import tpu_sc as plsc

_NBINS = 29
_NC, _NS, _L = 2, 16, 16
_NW = _NC * _NS
_LSTR = 33
_RSTR = 545
_QUAD = 4


@functools.partial(jax.jit, static_argnums=(3, 4))
def _hist_call(sim2, dtok, qtok, c_per_b, d):
    rows = sim2.shape[0]
    rpw = rows // _NW
    chunks = d // _L
    hsize = rpw * _RSTR

    mesh = plsc.VectorSubcoreMesh(core_axis_name="c", subcore_axis_name="s")

    @functools.partial(
        pl.kernel,
        mesh=mesh,
        compiler_params=pltpu.CompilerParams(needs_layout_passes=False),
        out_type=jax.ShapeDtypeStruct((rows, 32), jnp.float32),
        scratch_types=[
            pltpu.VMEM((rpw, d), jnp.float32),
            pltpu.VMEM((d,), jnp.int32),
            pltpu.VMEM((_L,), jnp.int32),
            pltpu.VMEM((d,), jnp.float32),
            pltpu.VMEM((_L,), jnp.float32),
            pltpu.VMEM((hsize,), jnp.float32),
            pltpu.VMEM((rpw, 32), jnp.float32),
            pltpu.SemaphoreType.DMA,
        ],
    )
    def body(sim_hbm, dtok_hbm, qtok_hbm, out_hbm, sim_v, dtok_v, qtok_v,
             wd_v, mq_v, hist_v, out_v, sem0):
        wid = lax.axis_index("s") * _NC + lax.axis_index("c")
        base = wid * rpw
        b = base // c_per_b

        cp0 = pltpu.async_copy(sim_hbm.at[pl.ds(base, rpw)], sim_v, sem0)
        pltpu.sync_copy(dtok_hbm.at[b], dtok_v)
        pltpu.sync_copy(qtok_hbm.at[b], qtok_v)

        zero = jnp.zeros((_L,), jnp.float32)
        one = jnp.ones((_L,), jnp.float32)

        @plsc.parallel_loop(0, hsize // _L, unroll=8)
        def _zbody(i):
            hist_v[pl.ds(i * _L, _L)] = zero

        @plsc.parallel_loop(0, chunks, unroll=4)
        def _wbody(i):
            t = dtok_v[pl.ds(i * _L, _L)]
            wd_v[pl.ds(i * _L, _L)] = jnp.where(t != jnp.int32(-1), one, zero)

        qt = qtok_v[pl.ds(0, _L)]
        mq_v[pl.ds(0, _L)] = jnp.where(qt != jnp.int32(-1), one, zero)

        lane = lax.iota(jnp.int32, _L) * _LSTR

        def quad_body(cq, carry):
            cb = cq * (_QUAD * _L)
            ws = [wd_v[pl.ds(cb + j * _L, _L)] for j in range(_QUAD)]

            @plsc.parallel_loop(0, rpw, unroll=4)
            def _rbody(r):
                lane_r = lane + r * _RSTR
                for j in range(_QUAD):
                    s = sim_v[r, pl.ds(cb + j * _L, _L)]
                    bins = ((s + 1.00001) * 14.0).astype(jnp.int32)
                    plsc.addupdate_scatter(hist_v, [bins + lane_r], ws[j])

            return carry

        cp0.wait()
        lax.fori_loop(0, chunks // _QUAD, quad_body, 0)

        def red_body(r, carry):
            @plsc.parallel_loop(
                0, _L, unroll=4,
                carry=(jnp.zeros((_L,), jnp.float32),
                       jnp.zeros((_L,), jnp.float32)))
            def _accs(ln, accs):
                a0, a1 = accs
                o = r * _RSTR + ln * _LSTR
                t0 = hist_v[pl.ds(o, _L)]
                t1 = hist_v[pl.ds(o + _L, _L)]
                hist_v[pl.ds(o, _L)] = zero
                hist_v[pl.ds(o + _L, _L)] = zero
                return (a0 + t0, a1 + t1)

            acc0, acc1 = _accs
            mq = plsc.load_gather(mq_v, [jnp.full((_L,), r % 16, jnp.int32)])
            out_v[r, pl.ds(0, _L)] = acc0 * mq
            out_v[r, pl.ds(_L, _L)] = acc1 * mq
            return carry

        lax.fori_loop(0, rpw, red_body, 0)

        pltpu.sync_copy(out_v, out_hbm.at[pl.ds(base, rpw)])

    return body(sim2, dtok, qtok)


def kernel(simmat, dlens, dtoks, qtoks):
    del dlens
    B, C, Q, D = simmat.shape
    sim2 = simmat.reshape(B * C * Q, D)
    dtok = dtoks.astype(jnp.int32)
    qtok = qtoks.astype(jnp.int32)
    out = _hist_call(sim2, dtok, qtok, C * Q, D)
    return out[:, :_NBINS].reshape(B, C, Q, _NBINS)


# speedup vs baseline: 1.0325x; 1.0179x over previous
import functools

import jax
import jax.numpy as jnp
from jax import lax
from jax.experimental import pallas as pl
from jax.experimental.pallas import tpu as pltpu
from jax.experimental.pallas import tpu_sc as plsc

_NBINS = 29
_NC, _NS, _L = 2, 16, 16
_NW = _NC * _NS
_LSTR = 33
_RSTR = 545
_QUAD = 4


@functools.partial(jax.jit, static_argnums=(3, 4))
def _hist_call(sim2, dtok, qtok, c_per_b, d):
    rows = sim2.shape[0]
    rpw = rows // _NW
    chunks = d // _L
    hsize = rpw * _RSTR

    mesh = plsc.VectorSubcoreMesh(core_axis_name="c", subcore_axis_name="s")

    @functools.partial(
        pl.kernel,
        mesh=mesh,
        compiler_params=pltpu.CompilerParams(needs_layout_passes=False),
        out_type=jax.ShapeDtypeStruct((rows, 32), jnp.float32),
        scratch_types=[
            pltpu.VMEM((rpw, d), jnp.float32),
            pltpu.VMEM((d,), jnp.int32),
            pltpu.VMEM((_L,), jnp.int32),
            pltpu.VMEM((d,), jnp.float32),
            pltpu.VMEM((_L,), jnp.float32),
            pltpu.VMEM((hsize,), jnp.float32),
            pltpu.VMEM((rpw, 32), jnp.float32),
            pltpu.SemaphoreType.DMA,
        ],
    )
    def body(sim_hbm, dtok_hbm, qtok_hbm, out_hbm, sim_v, dtok_v, qtok_v,
             wd_v, mq_v, hist_v, out_v, sem0):
        wid = lax.axis_index("s") * _NC + lax.axis_index("c")
        base = wid * rpw
        b = base // c_per_b

        cp0 = pltpu.async_copy(sim_hbm.at[pl.ds(base, rpw)], sim_v, sem0)
        pltpu.sync_copy(dtok_hbm.at[b], dtok_v)
        pltpu.sync_copy(qtok_hbm.at[b], qtok_v)

        zero = jnp.zeros((_L,), jnp.float32)
        one = jnp.ones((_L,), jnp.float32)

        @plsc.parallel_loop(0, hsize // _L, unroll=8)
        def _zbody(i):
            hist_v[pl.ds(i * _L, _L)] = zero

        @plsc.parallel_loop(0, chunks, unroll=4)
        def _wbody(i):
            t = dtok_v[pl.ds(i * _L, _L)]
            wd_v[pl.ds(i * _L, _L)] = jnp.where(t != jnp.int32(-1), one, zero)

        qt = qtok_v[pl.ds(0, _L)]
        mq_v[pl.ds(0, _L)] = jnp.where(qt != jnp.int32(-1), one, zero)

        lane = lax.iota(jnp.int32, _L) * _LSTR

        def quad_body(cq, carry):
            cb = cq * (_QUAD * _L)
            ws = [wd_v[pl.ds(cb + j * _L, _L)] for j in range(_QUAD)]

            @plsc.parallel_loop(0, rpw, unroll=8)
            def _rbody(r):
                lane_r = lane + r * _RSTR
                for j in range(_QUAD):
                    s = sim_v[r, pl.ds(cb + j * _L, _L)]
                    bins = ((s + 1.00001) * 14.0).astype(jnp.int32)
                    plsc.addupdate_scatter(hist_v, [bins + lane_r], ws[j])

            return carry

        cp0.wait()
        lax.fori_loop(0, chunks // _QUAD, quad_body, 0)

        def red_body(r, carry):
            @plsc.parallel_loop(
                0, _L, unroll=4,
                carry=(jnp.zeros((_L,), jnp.float32),
                       jnp.zeros((_L,), jnp.float32)))
            def _accs(ln, accs):
                a0, a1 = accs
                o = r * _RSTR + ln * _LSTR
                t0 = hist_v[pl.ds(o, _L)]
                t1 = hist_v[pl.ds(o + _L, _L)]
                hist_v[pl.ds(o, _L)] = zero
                hist_v[pl.ds(o + _L, _L)] = zero
                return (a0 + t0, a1 + t1)

            acc0, acc1 = _accs
            mq = plsc.load_gather(mq_v, [jnp.full((_L,), r % 16, jnp.int32)])
            out_v[r, pl.ds(0, _L)] = acc0 * mq
            out_v[r, pl.ds(_L, _L)] = acc1 * mq
            return carry

        lax.fori_loop(0, rpw, red_body, 0)

        pltpu.sync_copy(out_v, out_hbm.at[pl.ds(base, rpw)])

    return body(sim2, dtok, qtok)


def kernel(simmat, dlens, dtoks, qtoks):
    del dlens
    B, C, Q, D = simmat.shape
    sim2 = simmat.reshape(B * C * Q, D)
    dtok = dtoks.astype(jnp.int32)
    qtok = qtoks.astype(jnp.int32)
    out = _hist_call(sim2, dtok, qtok, C * Q, D)
    return out[:, :_NBINS].reshape(B, C, Q, _NBINS)
